# Initial kernel scaffold; baseline (speedup 1.0000x reference)
import jax
import jax.numpy as jnp
from jax.experimental import pallas as pl


def kernel(entity_emb, edge_index):
    raise NotImplementedError("write your pallas kernel here")


# speedup vs baseline: 5.1552x; 5.1552x over previous
import functools

import jax
import jax.numpy as jnp
from jax import lax
from jax.experimental import pallas as pl
from jax.experimental.pallas import tpu as pltpu
from jax.experimental.pallas import tpu_sc as plsc

N_NODES = 50000
N_REL = 4
E_REL = 200000
HALF = 32
NS = 16
CHUNK = 128
N_CHUNKS = 98
TILE_E = N_CHUNKS * CHUNK
E_PAD = NS * TILE_E
ACC_ROWS = 50176
FLUSH = 112
N_FLUSH = 28
TILE_R = N_FLUSH * FLUSH

_mesh = plsc.VectorSubcoreMesh(core_axis_name="c", subcore_axis_name="s")


@functools.partial(
    pl.kernel,
    out_type=(
        jax.ShapeDtypeStruct((N_NODES, 256), jnp.float32),
        jax.ShapeDtypeStruct((N_NODES, 64), jnp.float32),
        jax.ShapeDtypeStruct((N_NODES, 64), jnp.float32),
        jax.ShapeDtypeStruct((N_NODES, 64), jnp.float32),
        jax.ShapeDtypeStruct((N_NODES, 64), jnp.float32),
    ),
    mesh=_mesh,
    compiler_params=pltpu.CompilerParams(use_tc_tiling_on_sc=False),
    scratch_types=[
        pltpu.VMEM_SHARED((ACC_ROWS, HALF), jnp.float32),
        pltpu.VMEM_SHARED((ACC_ROWS,), jnp.float32),
        pltpu.VMEM((N_CHUNKS // 2, CHUNK), jnp.int32),
        pltpu.VMEM((N_CHUNKS // 2, CHUNK), jnp.int32),
        pltpu.VMEM((CHUNK, HALF), jnp.float32),
        pltpu.VMEM((CHUNK,), jnp.float32),
        pltpu.VMEM((FLUSH, HALF), jnp.float32),
        pltpu.VMEM((FLUSH,), jnp.float32),
        pltpu.SemaphoreType.DMA,
    ],
)
def _sc_encode(emb_flat, src_hbm, dst_hbm,
               out_cat, out_m0, out_m1, out_m2, out_m3,
               acc, cnt, src_v, dst_v, rows_v, ones_v,
               sum_v, cnt_v, sem):
    c = lax.axis_index("c")
    t = lax.axis_index("s")
    out_means = (out_m0, out_m1, out_m2, out_m3)

    z16 = jnp.zeros((16,), jnp.float32)
    o16 = jnp.ones((16,), jnp.float32)

    for j in range(CHUNK // 16):
        ones_v[pl.ds(16 * j, 16)] = o16

    col_m = 32 * c

    for r in range(N_REL):
        soff = 2 * r + c
        soff_v = jnp.full((16,), soff, jnp.int32)
        col_c = 32 * soff

        def fill_zrow(i, carry):
            sum_v[i, pl.ds(0, 16)] = z16
            sum_v[i, pl.ds(16, 16)] = z16
            return carry
        lax.fori_loop(0, FLUSH, fill_zrow, None)
        for j in range(FLUSH // 16):
            cnt_v[pl.ds(16 * j, 16)] = z16

        def zero_body(g, carry):
            r0 = t * TILE_R + g * FLUSH
            pltpu.sync_copy(sum_v, acc.at[pl.ds(r0, FLUSH), :])
            pltpu.sync_copy(cnt_v, cnt.at[pl.ds(r0, FLUSH)])
            return carry
        lax.fori_loop(0, N_FLUSH, zero_body, None)

        plsc.subcore_barrier()

        for p in range(2):
            pltpu.sync_copy(src_hbm.at[r, t, p], src_v)
            pltpu.sync_copy(dst_hbm.at[r, t, p], dst_v)

            def xform_body(g, carry):
                for j in range(CHUNK // 16):
                    v = src_v[g, pl.ds(16 * j, 16)]
                    src_v[g, pl.ds(16 * j, 16)] = v * 8 + soff_v
                return carry
            lax.fori_loop(0, N_CHUNKS // 2, xform_body, None)

            def acc_body(g, carry):
                pltpu.async_copy(emb_flat.at[src_v.at[g]], rows_v, sem).wait()
                pltpu.sync_copy(rows_v, acc.at[dst_v.at[g]], add=True)
                pltpu.sync_copy(ones_v, cnt.at[dst_v.at[g]], add=True)
                return carry
            lax.fori_loop(0, N_CHUNKS // 2, acc_body, None)

        plsc.subcore_barrier()

        def flush_body(g, carry):
            r0 = t * TILE_R + g * FLUSH
            pltpu.sync_copy(acc.at[pl.ds(r0, FLUSH), :], sum_v)
            pltpu.sync_copy(cnt.at[pl.ds(r0, FLUSH)], cnt_v)

            def div_body(k, dcarry):
                c16 = cnt_v[pl.ds(16 * k, 16)]
                inv16 = 1.0 / jnp.maximum(c16, 1.0)
                for j in range(16):
                    i = 16 * k + j
                    invv = jnp.full((16,), inv16[j], jnp.float32)
                    sum_v[i, pl.ds(0, 16)] = sum_v[i, pl.ds(0, 16)] * invv
                    sum_v[i, pl.ds(16, 16)] = sum_v[i, pl.ds(16, 16)] * invv
                return dcarry
            lax.fori_loop(0, FLUSH // 16, div_body, None)

            @pl.when(r0 + FLUSH <= N_NODES)
            def _():
                pltpu.sync_copy(sum_v, out_cat.at[pl.ds(r0, FLUSH), pl.ds(col_c, HALF)])
                pltpu.sync_copy(sum_v, out_means[r].at[pl.ds(r0, FLUSH), pl.ds(col_m, HALF)])

            @pl.when(jnp.logical_and(r0 < N_NODES, r0 + FLUSH > N_NODES))
            def _():
                w = N_NODES % FLUSH
                pltpu.sync_copy(sum_v.at[pl.ds(0, w), :],
                                out_cat.at[pl.ds(r0, w), pl.ds(col_c, HALF)])
                pltpu.sync_copy(sum_v.at[pl.ds(0, w), :],
                                out_means[r].at[pl.ds(r0, w), pl.ds(col_m, HALF)])
            return carry
        lax.fori_loop(0, N_FLUSH, flush_body, None)

        plsc.subcore_barrier()


def kernel(entity_emb, edge_index):
    ei = edge_index.astype(jnp.int32)
    src = ei[0].reshape(N_REL, E_REL)
    dst = ei[1].reshape(N_REL, E_REL)
    pad = E_PAD - E_REL
    src_p = jnp.pad(src, ((0, 0), (0, pad)))
    dst_p = jnp.pad(dst, ((0, 0), (0, pad)), constant_values=N_NODES)
    src_p = src_p.reshape(N_REL, NS, 2, N_CHUNKS // 2, CHUNK)
    dst_p = dst_p.reshape(N_REL, NS, 2, N_CHUNKS // 2, CHUNK)
    emb_flat = entity_emb.reshape(N_NODES * 8, HALF)
    out_cat, m0, m1, m2, m3 = _sc_encode(emb_flat, src_p, dst_p)
    return (out_cat, m0, m1, m2, m3)
